# baseline (speedup 1.0000x reference)
import functools

import jax
import jax.numpy as jnp
from jax import lax
from jax.experimental import pallas as pl
from jax.experimental.pallas import tpu as pltpu
from jax.experimental.pallas import tpu_sc as plsc

_BATCH = 4096
_SEQ = 200
_ND = 64
_NPART = 8
_NW = 32
_BB = _BATCH // _NW
_NBG = _BB // 16


def _compute_one(iota, parity, tbuf, ppbuf, obuf, rows, pidx64):
    @plsc.parallel_loop(0, _ND, unroll=8)
    def d_body(i):
        dv = ((iota + i) & 15) + (i & ~15)
        dtv = dv >> 3
        inv = (dv & 7) << 7
        for g in range(_NBG):
            cv = plsc.load_gather(tbuf.at[parity], [rows[g], dv])
            pv = plsc.load_gather(ppbuf, [pidx64[g] + dv])
            plsc.store_scatter(obuf.at[parity], [dtv, inv + rows[g]], cv + pv)


def _sc_body(cid_t, part_t, pos_hbm, content_hbm, part_hbm, out_hbm,
             cidl, partl, posl, part8, ppbuf, tbuf, obuf,
             semg, semo, semi):
    wid = lax.axis_index("s") * 2 + lax.axis_index("c")
    col0 = wid * _BB

    pltpu.sync_copy(cid_t.at[:, pl.ds(col0, _BB)], cidl)
    pltpu.sync_copy(part_t.at[:, pl.ds(col0, _BB)], partl)
    pltpu.sync_copy(pos_hbm, posl)
    pltpu.sync_copy(part_hbm, part8)

    iota = lax.iota(jnp.int32, 16)
    rows = [iota + (16 * g) for g in range(_NBG)]

    def fire_gather(s_row, parity):
        return pltpu.async_copy(content_hbm.at[cidl.at[s_row]],
                                tbuf.at[parity], semg)

    def wait_gather(s_row, parity):
        pltpu.make_async_copy(content_hbm.at[cidl.at[s_row]],
                              tbuf.at[parity], semg).wait()

    def fire_out(s, parity):
        return pltpu.async_copy(obuf.at[parity], out_hbm.at[s, :, wid], semo)

    def wait_out(s, parity):
        pltpu.make_async_copy(obuf.at[parity], out_hbm.at[s, :, wid],
                              semo).wait()

    def prep_pp(s):
        @plsc.parallel_loop(0, _NPART)
        def p_body(p):
            for c in range(_ND // 16):
                sl = pl.ds(16 * c, 16)
                ppbuf[pl.ds(p * _ND + 16 * c, 16)] = posl[s, sl] + part8[p, sl]

    def load_pidx64(s):
        return [partl[s, pl.ds(16 * g, 16)] << 6 for g in range(_NBG)]

    fire_gather(0, 0)

    def pair_body(j, carry):
        s0 = 2 * j
        s1 = s0 + 1

        @pl.when(j > 0)
        def _():
            wait_out(s0 - 2, 0)
            wait_out(s1 - 2, 1)

        wait_gather(s0, 0)
        fire_gather(s1, 1)
        prep_pp(s0)
        _compute_one(iota, 0, tbuf, ppbuf, obuf, rows, load_pidx64(s0))
        fire_out(s0, 0)

        wait_gather(s1, 1)

        @pl.when(j < _SEQ // 2 - 1)
        def _():
            fire_gather(s0 + 2, 0)

        prep_pp(s1)
        _compute_one(iota, 1, tbuf, ppbuf, obuf, rows, load_pidx64(s1))
        fire_out(s1, 1)
        return carry

    lax.fori_loop(0, _SEQ // 2, pair_body, 0)
    wait_out(_SEQ - 2, 0)
    wait_out(_SEQ - 1, 1)


_mesh = plsc.VectorSubcoreMesh(core_axis_name="c", subcore_axis_name="s")

_gather_sum = functools.partial(
    pl.kernel,
    out_type=jax.ShapeDtypeStruct((_SEQ, _ND // 8, _NW, 8 * _BB), jnp.float32),
    mesh=_mesh,
    scratch_types=[
        pltpu.VMEM((_SEQ, _BB), jnp.int32),
        pltpu.VMEM((_SEQ, _BB), jnp.int32),
        pltpu.VMEM((_SEQ, _ND), jnp.float32),
        pltpu.VMEM((_NPART, _ND), jnp.float32),
        pltpu.VMEM((_NPART * _ND,), jnp.float32),
        pltpu.VMEM((2, _BB, _ND), jnp.float32),
        pltpu.VMEM((2, _ND // 8, 8 * _BB), jnp.float32),
        pltpu.SemaphoreType.DMA,
        pltpu.SemaphoreType.DMA,
        pltpu.SemaphoreType.DMA,
    ],
    compiler_params=pltpu.CompilerParams(use_tc_tiling_on_sc=False,
                                         needs_layout_passes=False),
)(_sc_body)


def kernel(content_id, part_id, position_table, content_table, part_table):
    cid_t = content_id.astype(jnp.int32).T
    part_t = part_id.astype(jnp.int32).T
    out5d = _gather_sum(cid_t, part_t, position_table, content_table,
                        part_table)
    out5d = out5d.reshape(_SEQ, _ND // 8, _NW, 8, _BB)
    return jnp.transpose(out5d, (2, 4, 0, 1, 3)).reshape(_BATCH, _SEQ, _ND)


# speedup vs baseline: 1.0907x; 1.0907x over previous
import functools

import jax
import jax.numpy as jnp
from jax import lax
from jax.experimental import pallas as pl
from jax.experimental.pallas import tpu as pltpu
from jax.experimental.pallas import tpu_sc as plsc

_BATCH = 4096
_SEQ = 200
_ND = 64
_NPART = 8
_NW = 32
_BB = _BATCH // _NW
_NBG = _BB // 16


def _compute_one(iota, parity, tbuf, ppbuf, obuf, rows, pidx64):
    @plsc.parallel_loop(0, _ND, unroll=4)
    def d_body(i):
        dv = ((iota + i) & 15) + (i & ~15)
        dtv = dv >> 3
        inv = (dv & 7) << 7
        for g in range(_NBG):
            cv = plsc.load_gather(tbuf.at[parity], [rows[g], dv])
            pv = plsc.load_gather(ppbuf, [pidx64[g] + dv])
            plsc.store_scatter(obuf.at[parity], [dtv, inv + rows[g]], cv + pv)


def _sc_body(cid_t, part_t, pos_hbm, content_hbm, part_hbm, out_hbm,
             cidl, partl, posl, part8, ppbuf, tbuf, obuf,
             semg, semo, semi):
    wid = lax.axis_index("s") * 2 + lax.axis_index("c")
    col0 = wid * _BB

    pltpu.sync_copy(cid_t.at[:, pl.ds(col0, _BB)], cidl)
    pltpu.sync_copy(part_t.at[:, pl.ds(col0, _BB)], partl)
    pltpu.sync_copy(pos_hbm, posl)
    pltpu.sync_copy(part_hbm, part8)

    iota = lax.iota(jnp.int32, 16)
    rows = [iota + (16 * g) for g in range(_NBG)]

    def fire_gather(s_row, parity):
        return pltpu.async_copy(content_hbm.at[cidl.at[s_row]],
                                tbuf.at[parity], semg)

    def wait_gather(s_row, parity):
        pltpu.make_async_copy(content_hbm.at[cidl.at[s_row]],
                              tbuf.at[parity], semg).wait()

    def fire_out(s, parity):
        return pltpu.async_copy(obuf.at[parity], out_hbm.at[s, :, wid], semo)

    def wait_out(s, parity):
        pltpu.make_async_copy(obuf.at[parity], out_hbm.at[s, :, wid],
                              semo).wait()

    def prep_pp(s):
        @plsc.parallel_loop(0, _NPART)
        def p_body(p):
            for c in range(_ND // 16):
                sl = pl.ds(16 * c, 16)
                ppbuf[pl.ds(p * _ND + 16 * c, 16)] = posl[s, sl] + part8[p, sl]

    def load_pidx64(s):
        return [partl[s, pl.ds(16 * g, 16)] << 6 for g in range(_NBG)]

    fire_gather(0, 0)

    def pair_body(j, carry):
        s0 = 2 * j
        s1 = s0 + 1

        prep_pp(s0)
        px0 = load_pidx64(s0)

        @pl.when(j > 0)
        def _():
            wait_out(s0 - 2, 0)
            wait_out(s1 - 2, 1)

        wait_gather(s0, 0)
        fire_gather(s1, 1)
        _compute_one(iota, 0, tbuf, ppbuf, obuf, rows, px0)
        fire_out(s0, 0)

        prep_pp(s1)
        px1 = load_pidx64(s1)
        wait_gather(s1, 1)

        @pl.when(j < _SEQ // 2 - 1)
        def _():
            fire_gather(s0 + 2, 0)

        _compute_one(iota, 1, tbuf, ppbuf, obuf, rows, px1)
        fire_out(s1, 1)
        return carry

    lax.fori_loop(0, _SEQ // 2, pair_body, 0)
    wait_out(_SEQ - 2, 0)
    wait_out(_SEQ - 1, 1)


_mesh = plsc.VectorSubcoreMesh(core_axis_name="c", subcore_axis_name="s")

_gather_sum = functools.partial(
    pl.kernel,
    out_type=jax.ShapeDtypeStruct((_SEQ, _ND // 8, _NW, 8 * _BB), jnp.float32),
    mesh=_mesh,
    scratch_types=[
        pltpu.VMEM((_SEQ, _BB), jnp.int32),
        pltpu.VMEM((_SEQ, _BB), jnp.int32),
        pltpu.VMEM((_SEQ, _ND), jnp.float32),
        pltpu.VMEM((_NPART, _ND), jnp.float32),
        pltpu.VMEM((_NPART * _ND,), jnp.float32),
        pltpu.VMEM((2, _BB, _ND), jnp.float32),
        pltpu.VMEM((2, _ND // 8, 8 * _BB), jnp.float32),
        pltpu.SemaphoreType.DMA,
        pltpu.SemaphoreType.DMA,
        pltpu.SemaphoreType.DMA,
    ],
    compiler_params=pltpu.CompilerParams(use_tc_tiling_on_sc=False,
                                         needs_layout_passes=False),
)(_sc_body)


def kernel(content_id, part_id, position_table, content_table, part_table):
    cid_t = content_id.astype(jnp.int32).T
    part_t = part_id.astype(jnp.int32).T
    out5d = _gather_sum(cid_t, part_t, position_table, content_table,
                        part_table)
    out5d = out5d.reshape(_SEQ, _ND // 8, _NW, 8, _BB)
    return jnp.transpose(out5d, (2, 4, 0, 1, 3)).reshape(_BATCH, _SEQ, _ND)


# speedup vs baseline: 1.1157x; 1.0229x over previous
import functools

import jax
import jax.numpy as jnp
from jax import lax
from jax.experimental import pallas as pl
from jax.experimental.pallas import tpu as pltpu
from jax.experimental.pallas import tpu_sc as plsc

_BATCH = 4096
_SEQ = 200
_ND = 64
_NPART = 8
_NW = 32
_BB = _BATCH // _NW
_NBG = _BB // 16


def _compute_one(iota, tslot, parity, tbuf, ppbuf, obuf, rows, pidx64):
    @plsc.parallel_loop(0, _ND, unroll=4)
    def d_body(i):
        dv = ((iota + i) & 15) + (i & ~15)
        dtv = dv >> 3
        inv = (dv & 7) << 7
        for g in range(_NBG):
            cv = plsc.load_gather(tbuf.at[tslot], [rows[g], dv])
            pv = plsc.load_gather(ppbuf, [pidx64[g] + dv])
            plsc.store_scatter(obuf.at[parity], [dtv, inv + rows[g]], cv + pv)


def _sc_body(cid_t, part_t, pos_hbm, content_hbm, part_hbm, out_hbm,
             cidl, partl, posl, part8, ppbuf, tbuf, obuf,
             semg, semo, semi):
    wid = lax.axis_index("s") * 2 + lax.axis_index("c")
    col0 = wid * _BB

    pltpu.sync_copy(cid_t.at[:, pl.ds(col0, _BB)], cidl)
    pltpu.sync_copy(part_t.at[:, pl.ds(col0, _BB)], partl)
    pltpu.sync_copy(pos_hbm, posl)
    pltpu.sync_copy(part_hbm, part8)

    iota = lax.iota(jnp.int32, 16)
    rows = [iota + (16 * g) for g in range(_NBG)]

    def fire_gather(s_row, parity):
        return pltpu.async_copy(content_hbm.at[cidl.at[s_row]],
                                tbuf.at[parity], semg)

    def wait_gather(s_row, parity):
        pltpu.make_async_copy(content_hbm.at[cidl.at[s_row]],
                              tbuf.at[parity], semg).wait()

    def fire_out(s, parity):
        return pltpu.async_copy(obuf.at[parity], out_hbm.at[s, :, wid], semo)

    def wait_out(s, parity):
        pltpu.make_async_copy(obuf.at[parity], out_hbm.at[s, :, wid],
                              semo).wait()

    def prep_pp(s):
        @plsc.parallel_loop(0, _NPART)
        def p_body(p):
            for c in range(_ND // 16):
                sl = pl.ds(16 * c, 16)
                ppbuf[pl.ds(p * _ND + 16 * c, 16)] = posl[s, sl] + part8[p, sl]

    def load_pidx64(s):
        return [partl[s, pl.ds(16 * g, 16)] << 6 for g in range(_NBG)]

    fire_gather(0, 0)
    fire_gather(1, 1)
    fire_gather(2, 2)

    def quad_body(j, carry):
        for k in range(4):
            s = 4 * j + k
            parity = k % 2
            prep_pp(s)
            px = load_pidx64(s)

            @pl.when(s >= 2)
            def _():
                wait_out(s - 2, parity)

            wait_gather(s, k)

            @pl.when(s + 3 < _SEQ)
            def _():
                fire_gather(s + 3, (k + 3) % 4)

            _compute_one(iota, k, parity, tbuf, ppbuf, obuf, rows, px)
            fire_out(s, parity)
        return carry

    lax.fori_loop(0, _SEQ // 4, quad_body, 0)
    wait_out(_SEQ - 2, 0)
    wait_out(_SEQ - 1, 1)


_mesh = plsc.VectorSubcoreMesh(core_axis_name="c", subcore_axis_name="s")

_gather_sum = functools.partial(
    pl.kernel,
    out_type=jax.ShapeDtypeStruct((_SEQ, _ND // 8, _NW, 8 * _BB), jnp.float32),
    mesh=_mesh,
    scratch_types=[
        pltpu.VMEM((_SEQ, _BB), jnp.int32),
        pltpu.VMEM((_SEQ, _BB), jnp.int32),
        pltpu.VMEM((_SEQ, _ND), jnp.float32),
        pltpu.VMEM((_NPART, _ND), jnp.float32),
        pltpu.VMEM((_NPART * _ND,), jnp.float32),
        pltpu.VMEM((4, _BB, _ND), jnp.float32),
        pltpu.VMEM((2, _ND // 8, 8 * _BB), jnp.float32),
        pltpu.SemaphoreType.DMA,
        pltpu.SemaphoreType.DMA,
        pltpu.SemaphoreType.DMA,
    ],
    compiler_params=pltpu.CompilerParams(use_tc_tiling_on_sc=False,
                                         needs_layout_passes=False),
)(_sc_body)


def kernel(content_id, part_id, position_table, content_table, part_table):
    cid_t = content_id.astype(jnp.int32).T
    part_t = part_id.astype(jnp.int32).T
    out5d = _gather_sum(cid_t, part_t, position_table, content_table,
                        part_table)
    out5d = out5d.reshape(_SEQ, _ND // 8, _NW, 8, _BB)
    return jnp.transpose(out5d, (2, 4, 0, 1, 3)).reshape(_BATCH, _SEQ, _ND)
